# baseline (speedup 1.0000x reference)
import jax
import jax.numpy as jnp
from jax.experimental import pallas as pl
from jax.experimental.pallas import tpu as pltpu

_K = 16


def _bf16_split3(v):
    hi = v.astype(jnp.bfloat16)
    r = v - hi.astype(jnp.float32)
    mid = r.astype(jnp.bfloat16)
    lo = (r - mid.astype(jnp.float32)).astype(jnp.bfloat16)
    return hi, mid, lo


def _augment_t(pt, norm_first):
    P = pt.shape[1]
    bf = jnp.bfloat16
    nrm = (pt[0:1] * pt[0:1] + pt[1:2] * pt[1:2]) + pt[2:3] * pt[2:3]
    hi, mid, lo = _bf16_split3(nrm)
    ones = jnp.ones((3, P), bf)
    zeros = jnp.zeros((_K - 9, P), bf)
    if norm_first:
        pieces = [(-2.0 * pt).astype(bf), hi, mid, lo, ones, zeros]
    else:
        pieces = [pt.astype(bf), ones, hi, mid, lo, zeros]
    return jnp.concatenate(pieces, axis=0)


def _chamfer_body(x_ref, y_ref, loss_ref, lsum_ref):
    b = pl.program_id(0)
    nb = pl.num_programs(0)

    xat = _augment_t(x_ref[0].T, True)
    ya = _augment_t(y_ref[0].T, False)

    dist = jax.lax.dot_general(
        xat, ya, (((0,), (0,)), ((), ())),
        preferred_element_type=jnp.float32)

    row_min = jnp.maximum(jnp.min(dist, axis=1), 0.0)
    col_min = jnp.maximum(jnp.min(dist, axis=0), 0.0)

    @pl.when(b == 0)
    def _():
        lsum_ref[0, 0] = 0.0

    lsum_ref[0, 0] += jnp.sum(row_min) + jnp.sum(col_min)

    @pl.when(b == nb - 1)
    def _():
        loss_ref[0, 0] = lsum_ref[0, 0] * (1.0 / nb)


def kernel(x, y):
    B, N, _ = x.shape
    M = y.shape[1]

    loss = pl.pallas_call(
        _chamfer_body,
        grid=(B,),
        in_specs=[
            pl.BlockSpec((1, N, 3), lambda b: (b, 0, 0)),
            pl.BlockSpec((1, M, 3), lambda b: (b, 0, 0)),
        ],
        out_specs=pl.BlockSpec(
            (1, 1), lambda b: (0, 0), memory_space=pltpu.SMEM),
        out_shape=jax.ShapeDtypeStruct((1, 1), jnp.float32),
        scratch_shapes=[pltpu.SMEM((1, 1), jnp.float32)],
    )(x, y)
    return jnp.reshape(loss, ())


# speedup vs baseline: 1.5062x; 1.5062x over previous
import jax
import jax.numpy as jnp
from jax.experimental import pallas as pl
from jax.experimental.pallas import tpu as pltpu

_K = 16


def _bf16_split3(v):
    hi = v.astype(jnp.bfloat16)
    r = v - hi.astype(jnp.float32)
    mid = r.astype(jnp.bfloat16)
    lo = (r - mid.astype(jnp.float32)).astype(jnp.bfloat16)
    return hi, mid, lo


def _augment_t(pt, norm_first):
    P = pt.shape[1]
    bf = jnp.bfloat16
    nrm = (pt[0:1] * pt[0:1] + pt[1:2] * pt[1:2]) + pt[2:3] * pt[2:3]
    hi, mid, lo = _bf16_split3(nrm)
    ones = jnp.ones((3, P), bf)
    zeros = jnp.zeros((_K - 9, P), bf)
    if norm_first:
        pieces = [(-2.0 * pt).astype(bf), hi, mid, lo, ones, zeros]
    else:
        pieces = [pt.astype(bf), ones, hi, mid, lo, zeros]
    return jnp.concatenate(pieces, axis=0)


def _chamfer_body(xt_ref, yt_ref, loss_ref, lsum_ref):
    b = pl.program_id(0)
    nb = pl.num_programs(0)

    xat = _augment_t(xt_ref[0], True)
    ya = _augment_t(yt_ref[0], False)

    dist = jax.lax.dot_general(
        xat, ya, (((0,), (0,)), ((), ())),
        preferred_element_type=jnp.float32)

    row_min = jnp.maximum(jnp.min(dist, axis=1), 0.0)
    col_min = jnp.maximum(jnp.min(dist, axis=0), 0.0)

    @pl.when(b == 0)
    def _():
        lsum_ref[0, 0] = 0.0

    lsum_ref[0, 0] += jnp.sum(row_min) + jnp.sum(col_min)

    @pl.when(b == nb - 1)
    def _():
        loss_ref[0, 0] = lsum_ref[0, 0] * (1.0 / nb)


def kernel(x, y):
    B, N, _ = x.shape
    M = y.shape[1]
    xyt = jnp.swapaxes(jnp.concatenate([x, y], axis=1), 1, 2)

    loss = pl.pallas_call(
        _chamfer_body,
        grid=(B,),
        in_specs=[
            pl.BlockSpec((1, 3, N), lambda b: (b, 0, 0)),
            pl.BlockSpec((1, 3, M), lambda b: (b, 0, 1)),
        ],
        out_specs=pl.BlockSpec(
            (1, 1), lambda b: (0, 0), memory_space=pltpu.SMEM),
        out_shape=jax.ShapeDtypeStruct((1, 1), jnp.float32),
        scratch_shapes=[pltpu.SMEM((1, 1), jnp.float32)],
    )(xyt, xyt)
    return jnp.reshape(loss, ())


# speedup vs baseline: 1.6543x; 1.0983x over previous
import jax
import jax.numpy as jnp
from jax.experimental import pallas as pl
from jax.experimental.pallas import tpu as pltpu

_K = 16


def _bf16_split3(v):
    hi = v.astype(jnp.bfloat16)
    r = v - hi.astype(jnp.float32)
    mid = r.astype(jnp.bfloat16)
    lo = (r - mid.astype(jnp.float32)).astype(jnp.bfloat16)
    return hi, mid, lo


def _augment_t(pt, norm_first):
    P = pt.shape[1]
    bf = jnp.bfloat16
    nrm = (pt[0:1] * pt[0:1] + pt[1:2] * pt[1:2]) + pt[2:3] * pt[2:3]
    hi, mid, lo = _bf16_split3(nrm)
    ones = jnp.ones((3, P), bf)
    zeros = jnp.zeros((_K - 9, P), bf)
    if norm_first:
        pieces = [(-2.0 * pt).astype(bf), hi, mid, lo, ones, zeros]
    else:
        pieces = [pt.astype(bf), ones, hi, mid, lo, zeros]
    return jnp.concatenate(pieces, axis=0)


def _chamfer_body(xt_ref, yt_ref, loss_ref):
    nb = xt_ref.shape[0]
    total = jnp.float32(0.0)
    for b in range(nb):
        xat = _augment_t(xt_ref[b], True)
        ya = _augment_t(yt_ref[b], False)
        dist = jax.lax.dot_general(
            xat, ya, (((0,), (0,)), ((), ())),
            preferred_element_type=jnp.float32)
        row_min = jnp.maximum(jnp.min(dist, axis=1), 0.0)
        col_min = jnp.maximum(jnp.min(dist, axis=0), 0.0)
        total += jnp.sum(row_min) + jnp.sum(col_min)
    loss_ref[0, 0] = total * (1.0 / nb)


def kernel(x, y):
    B, N, _ = x.shape
    M = y.shape[1]
    xt = jnp.swapaxes(x, 1, 2)
    yt = jnp.swapaxes(y, 1, 2)

    loss = pl.pallas_call(
        _chamfer_body,
        grid=(1,),
        in_specs=[
            pl.BlockSpec((B, 3, N), lambda i: (0, 0, 0)),
            pl.BlockSpec((B, 3, M), lambda i: (0, 0, 0)),
        ],
        out_specs=pl.BlockSpec(
            (1, 1), lambda i: (0, 0), memory_space=pltpu.SMEM),
        out_shape=jax.ShapeDtypeStruct((1, 1), jnp.float32),
    )(xt, yt)
    return jnp.reshape(loss, ())
